# baseline (speedup 1.0000x reference)
import jax
import jax.numpy as jnp
from jax import lax
from jax.experimental import pallas as pl
from jax.experimental.pallas import tpu as pltpu
from jax.experimental.pallas import tpu_sc as plsc

PRE_SEQ_LEN = 128
HIDDEN = 1024
NUM_LAYERS = 24
OUT_DIM = NUM_LAYERS * 2 * HIDDEN
BATCH = 4

NB = BATCH * PRE_SEQ_LEN
SPLIT = 32
DC = OUT_DIM // SPLIT
NBUF = 4

NC, NS = 2, 16
NW = NC * NS
B_PER_W = NB // NW


def _body(table, idx_hbm, out, idx_v, buf0, buf1, buf2, buf3,
          gsem0, gsem1, gsem2, gsem3, wsem0, wsem1, wsem2, wsem3):
    wid = lax.axis_index("s") * NC + lax.axis_index("c")
    base = wid * B_PER_W

    bufs = (buf0, buf1, buf2, buf3)
    gsems = (gsem0, gsem1, gsem2, gsem3)
    wsems = (wsem0, wsem1, wsem2, wsem3)

    pltpu.sync_copy(idx_hbm.at[pl.ds(base, B_PER_W)], idx_v)

    gd = [None] * NBUF
    wd = [None] * NBUF
    for c in range(SPLIT + 1):
        if c < SPLIT:
            b = c % NBUF
            if wd[b] is not None:
                wd[b].wait()
            gd[b] = pltpu.async_copy(
                table.at[idx_v, pl.ds(c * DC, DC)], bufs[b], gsems[b]
            )
        if c >= 1:
            b = (c - 1) % NBUF
            gd[b].wait()
            wd[b] = pltpu.async_copy(
                bufs[b],
                out.at[pl.ds(base, B_PER_W), pl.ds((c - 1) * DC, DC)],
                wsems[b],
            )
    for b in range(NBUF):
        wd[b].wait()


@jax.jit
def _sc_gather(table, idx):
    mesh = plsc.VectorSubcoreMesh(core_axis_name="c", subcore_axis_name="s")
    k = pl.kernel(
        _body,
        out_type=jax.ShapeDtypeStruct((NB, OUT_DIM), jnp.float32),
        mesh=mesh,
        scratch_types=(
            [pltpu.VMEM((B_PER_W,), jnp.int32)]
            + [pltpu.VMEM((B_PER_W, DC), jnp.float32)] * NBUF
            + [pltpu.SemaphoreType.DMA] * (2 * NBUF)
        ),
    )
    return k(table, idx)


def kernel(prefix, embedding_weight):
    idx = prefix.reshape(NB)
    out = _sc_gather(embedding_weight, idx)
    return out.reshape(BATCH, PRE_SEQ_LEN, OUT_DIM)
